# baseline (speedup 1.0000x reference)
import jax
import jax.numpy as jnp
from jax.experimental import pallas as pl

_BATCH_BLOCK = 8


def _body(fp_ref, base_ref, d_ref, out_ref):
    f = fp_ref[...].astype(jnp.float32)
    dl = d_ref[0]
    out_ref[...] = (base_ref[...][None, :, :]
                    + f[:, :, None] * dl[None, None, :])


def kernel(fp, pair_emb, bit_emb, val_emb):
    B, D = fp.shape
    E = val_emb.shape[1]
    H = D // 2
    base = (jnp.repeat(pair_emb, 2, axis=0)
            + jnp.tile(bit_emb, (H, 1))
            + val_emb[0][None, :])
    delta = (val_emb[1] - val_emb[0])[None, :]
    out = pl.pallas_call(
        _body,
        grid=(B // _BATCH_BLOCK,),
        in_specs=[
            pl.BlockSpec((_BATCH_BLOCK, D), lambda i: (i, 0)),
            pl.BlockSpec((D, E), lambda i: (0, 0)),
            pl.BlockSpec((1, E), lambda i: (0, 0)),
        ],
        out_specs=pl.BlockSpec((_BATCH_BLOCK, D, E), lambda i: (i, 0, 0)),
        out_shape=jax.ShapeDtypeStruct((B, D, E), jnp.float32),
    )(fp, base, delta)
    return out


# speedup vs baseline: 5.8573x; 5.8573x over previous
import jax
import jax.numpy as jnp
from jax.experimental import pallas as pl

_BATCH_BLOCK = 8


def _body(fp_ref, baset_ref, deltat_ref, out_ref):
    f = fp_ref[...].astype(jnp.float32)
    out_ref[...] = (baset_ref[...][None, :, :]
                    + f[:, None, :] * deltat_ref[...][None, :, :])


def kernel(fp, pair_emb, bit_emb, val_emb):
    B, D = fp.shape
    E = val_emb.shape[1]
    H = D // 2
    base = (jnp.repeat(pair_emb, 2, axis=0)
            + jnp.tile(bit_emb, (H, 1))
            + val_emb[0][None, :])
    baset = base.T
    deltat = jnp.broadcast_to((val_emb[1] - val_emb[0])[:, None], (E, D))
    outt = pl.pallas_call(
        _body,
        grid=(B // _BATCH_BLOCK,),
        in_specs=[
            pl.BlockSpec((_BATCH_BLOCK, D), lambda i: (i, 0)),
            pl.BlockSpec((E, D), lambda i: (0, 0)),
            pl.BlockSpec((E, D), lambda i: (0, 0)),
        ],
        out_specs=pl.BlockSpec((_BATCH_BLOCK, E, D), lambda i: (i, 0, 0)),
        out_shape=jax.ShapeDtypeStruct((B, E, D), jnp.float32),
    )(fp, baset, deltat)
    return jnp.swapaxes(outt, 1, 2)
